# Initial kernel scaffold; baseline (speedup 1.0000x reference)
import jax
import jax.numpy as jnp
from jax.experimental import pallas as pl


def kernel(x, edge_index, edge_attr):
    raise NotImplementedError("write your pallas kernel here")


# speedup vs baseline: 17.8096x; 17.8096x over previous
import functools

import jax
import jax.numpy as jnp
from jax import lax
from jax.experimental import pallas as pl
from jax.experimental.pallas import tpu as pltpu
from jax.experimental.pallas import tpu_sc as plsc

_N_NODES = 10000
_N_EDGES = 320000
_NC = 2
_NS = 16
_NW = _NC * _NS
_E_W = _N_EDGES // _NW
_VECS = _E_W // 16
_AR = 80
_NPAD = _AR * 128


def _sc_partials(u, src, dst, attr_flat, rows):
    mesh = plsc.VectorSubcoreMesh(
        core_axis_name="c", subcore_axis_name="s",
        num_cores=_NC, num_subcores=_NS)

    @functools.partial(
        pl.kernel,
        mesh=mesh,
        out_type=jax.ShapeDtypeStruct((_NC, 3, _AR, 128), jnp.float32),
        compiler_params=pltpu.CompilerParams(needs_layout_passes=False),
        scratch_types=[
            pltpu.VMEM((_N_NODES,), jnp.float32),
            pltpu.VMEM((_E_W,), jnp.int32),
            pltpu.VMEM((_E_W,), jnp.int32),
            pltpu.VMEM((_E_W * 4,), jnp.float32),
            pltpu.VMEM((_AR, 128), jnp.float32),
            pltpu.VMEM((_AR, 128), jnp.float32),
            pltpu.VMEM((_AR, 128), jnp.float32),
            pltpu.VMEM((1, _AR), jnp.int32),
            pltpu.VMEM_SHARED((_AR, 128), jnp.float32),
            pltpu.VMEM_SHARED((_AR, 128), jnp.float32),
            pltpu.VMEM_SHARED((_AR, 128), jnp.float32),
        ],
    )
    def k(u_hbm, src_hbm, dst_hbm, attr_hbm, rows_hbm, out_hbm,
          u_v, src_v, dst_v, attr_v, accx, accy, accc, rows_v,
          shx, shy, shc):
        cid = lax.axis_index("c")
        sid = lax.axis_index("s")
        base = (cid * _NS + sid) * _E_W

        z16 = jnp.zeros((16,), jnp.float32)

        def zero_body(i, carry):
            r = lax.shift_right_logical(i, 3)
            o = lax.bitwise_and(i, 7) * 16
            accx[r, pl.ds(o, 16)] = z16
            accy[r, pl.ds(o, 16)] = z16
            accc[r, pl.ds(o, 16)] = z16
            return carry

        lax.fori_loop(0, _AR * 8, zero_body, 0)

        @pl.when(sid == 0)
        def _():
            pltpu.sync_copy(accx, shx)
            pltpu.sync_copy(accy, shy)
            pltpu.sync_copy(accc, shc)

        plsc.subcore_barrier()

        pltpu.sync_copy(u_hbm, u_v)
        pltpu.sync_copy(src_hbm.at[pl.ds(base, _E_W)], src_v)
        pltpu.sync_copy(dst_hbm.at[pl.ds(base, _E_W)], dst_v)
        pltpu.sync_copy(attr_hbm.at[pl.ds(base * 4, _E_W * 4)], attr_v)
        pltpu.sync_copy(rows_hbm, rows_v)

        iota4 = lax.iota(jnp.int32, 16) * 4
        ones_f = jnp.ones((16,), jnp.float32)

        def body(i, carry):
            off = pl.multiple_of(i * 16, 16)
            ids = src_v[pl.ds(off, 16)]
            idd = dst_v[pl.ds(off, 16)]
            us = plsc.load_gather(u_v, [ids])
            ud = plsc.load_gather(u_v, [idd])
            du = ud - us
            e4 = off * 4 + iota4
            a0 = plsc.load_gather(attr_v, [e4])
            a1 = plsc.load_gather(attr_v, [e4 + 1])
            r = lax.shift_right_logical(idd, 7)
            c = lax.bitwise_and(idd, 127)
            plsc.addupdate_scatter(accx, [r, c], du / a0)
            plsc.addupdate_scatter(accy, [r, c], du / a1)
            plsc.addupdate_scatter(accc, [r, c], ones_f)
            return carry

        lax.fori_loop(0, _VECS, body, 0)

        pltpu.sync_copy(accx, shx.at[rows_v.at[0]], add=True)
        pltpu.sync_copy(accy, shy.at[rows_v.at[0]], add=True)
        pltpu.sync_copy(accc, shc.at[rows_v.at[0]], add=True)

        plsc.subcore_barrier()

        @pl.when(sid == 0)
        def _():
            pltpu.sync_copy(shx, out_hbm.at[cid, 0])
            pltpu.sync_copy(shy, out_hbm.at[cid, 1])
            pltpu.sync_copy(shc, out_hbm.at[cid, 2])

    return k(u, src, dst, attr_flat, rows)


def _combine(parts):
    def ck(p_ref, o_ref):
        p = p_ref[...]
        s = p[0] + p[1]
        num = s[0:2]
        cnt = jnp.maximum(s[2:3], 1.0)
        o_ref[...] = num / cnt

    return pl.pallas_call(
        ck,
        out_shape=jax.ShapeDtypeStruct((2, _NPAD), jnp.float32),
    )(parts)


def kernel(x, edge_index, edge_attr):
    u = x[:, 0]
    rows = jnp.arange(_AR, dtype=jnp.int32).reshape(1, _AR)
    parts = _sc_partials(u, edge_index[0], edge_index[1],
                         edge_attr.reshape(-1), rows)
    parts = parts.reshape(_NC, 3, _NPAD)
    o = _combine(parts)
    return o[:, :_N_NODES].T


# speedup vs baseline: 60.4697x; 3.3953x over previous
import functools

import jax
import jax.numpy as jnp
from jax import lax
from jax.experimental import pallas as pl
from jax.experimental.pallas import tpu as pltpu
from jax.experimental.pallas import tpu_sc as plsc

_N_NODES = 10000
_N_EDGES = 320000
_NC = 2
_NS = 16
_NW = _NC * _NS
_E_W = _N_EDGES // _NW
_VECS = _E_W // 16
_AR = 80
_NPAD = _AR * 128


def _sc_partials(u, src, dst, a0, a1, rows):
    mesh = plsc.VectorSubcoreMesh(
        core_axis_name="c", subcore_axis_name="s",
        num_cores=_NC, num_subcores=_NS)

    @functools.partial(
        pl.kernel,
        mesh=mesh,
        out_type=jax.ShapeDtypeStruct((_NC, 3, _AR, 128), jnp.float32),
        compiler_params=pltpu.CompilerParams(needs_layout_passes=False),
        scratch_types=[
            pltpu.VMEM((_N_NODES,), jnp.float32),
            pltpu.VMEM((_E_W,), jnp.int32),
            pltpu.VMEM((_E_W,), jnp.int32),
            pltpu.VMEM((_E_W,), jnp.float32),
            pltpu.VMEM((_E_W,), jnp.float32),
            pltpu.VMEM((_AR, 128), jnp.float32),
            pltpu.VMEM((_AR, 128), jnp.float32),
            pltpu.VMEM((_AR, 128), jnp.float32),
            pltpu.VMEM((1, _AR), jnp.int32),
            pltpu.VMEM_SHARED((_AR, 128), jnp.float32),
            pltpu.VMEM_SHARED((_AR, 128), jnp.float32),
            pltpu.VMEM_SHARED((_AR, 128), jnp.float32),
        ],
    )
    def k(u_hbm, src_hbm, dst_hbm, a0_hbm, a1_hbm, rows_hbm, out_hbm,
          u_v, src_v, dst_v, a0_v, a1_v, accx, accy, accc, rows_v,
          shx, shy, shc):
        cid = lax.axis_index("c")
        sid = lax.axis_index("s")
        base = (cid * _NS + sid) * _E_W

        z16 = jnp.zeros((16,), jnp.float32)

        def zero_body(i, carry):
            r = lax.shift_right_logical(i, 3)
            o = lax.bitwise_and(i, 7) * 16
            accx[r, pl.ds(o, 16)] = z16
            accy[r, pl.ds(o, 16)] = z16
            accc[r, pl.ds(o, 16)] = z16
            return carry

        lax.fori_loop(0, _AR * 8, zero_body, 0)

        @pl.when(sid == 0)
        def _():
            pltpu.sync_copy(accx, shx)
            pltpu.sync_copy(accy, shy)
            pltpu.sync_copy(accc, shc)

        plsc.subcore_barrier()

        pltpu.sync_copy(u_hbm, u_v)
        pltpu.sync_copy(src_hbm.at[pl.ds(base, _E_W)], src_v)
        pltpu.sync_copy(dst_hbm.at[pl.ds(base, _E_W)], dst_v)
        pltpu.sync_copy(a0_hbm.at[pl.ds(base, _E_W)], a0_v)
        pltpu.sync_copy(a1_hbm.at[pl.ds(base, _E_W)], a1_v)
        pltpu.sync_copy(rows_hbm, rows_v)

        ones_f = jnp.ones((16,), jnp.float32)

        def body(i, carry):
            off = pl.multiple_of(i * 16, 16)
            ids = src_v[pl.ds(off, 16)]
            idd = dst_v[pl.ds(off, 16)]
            us = plsc.load_gather(u_v, [ids])
            ud = plsc.load_gather(u_v, [idd])
            du = ud - us
            a0 = a0_v[pl.ds(off, 16)]
            a1 = a1_v[pl.ds(off, 16)]
            r = lax.shift_right_logical(idd, 7)
            c = lax.bitwise_and(idd, 127)
            plsc.addupdate_scatter(accx, [r, c], du / a0)
            plsc.addupdate_scatter(accy, [r, c], du / a1)
            plsc.addupdate_scatter(accc, [r, c], ones_f)
            return carry

        lax.fori_loop(0, _VECS, body, 0)

        pltpu.sync_copy(accx, shx.at[rows_v.at[0]], add=True)
        pltpu.sync_copy(accy, shy.at[rows_v.at[0]], add=True)
        pltpu.sync_copy(accc, shc.at[rows_v.at[0]], add=True)

        plsc.subcore_barrier()

        @pl.when(sid == 0)
        def _():
            pltpu.sync_copy(shx, out_hbm.at[cid, 0])
            pltpu.sync_copy(shy, out_hbm.at[cid, 1])
            pltpu.sync_copy(shc, out_hbm.at[cid, 2])

    return k(u, src, dst, a0, a1, rows)


def _combine(parts):
    def ck(p_ref, o_ref):
        p = p_ref[...]
        s = p[0] + p[1]
        num = s[0:2]
        cnt = jnp.maximum(s[2:3], 1.0)
        o_ref[...] = num / cnt

    return pl.pallas_call(
        ck,
        out_shape=jax.ShapeDtypeStruct((2, _NPAD), jnp.float32),
    )(parts)


def kernel(x, edge_index, edge_attr):
    u = x[:, 0]
    rows = jnp.arange(_AR, dtype=jnp.int32).reshape(1, _AR)
    parts = _sc_partials(u, edge_index[0], edge_index[1],
                         edge_attr[:, 0], edge_attr[:, 1], rows)
    parts = parts.reshape(_NC, 3, _NPAD)
    o = _combine(parts)
    return o[:, :_N_NODES].T


# speedup vs baseline: 69.9387x; 1.1566x over previous
import functools

import jax
import jax.numpy as jnp
from jax import lax
from jax.experimental import pallas as pl
from jax.experimental.pallas import tpu as pltpu
from jax.experimental.pallas import tpu_sc as plsc

_N_NODES = 10000
_N_EDGES = 320000
_NC = 2
_NS = 16
_NW = _NC * _NS
_E_W = _N_EDGES // _NW
_VECS = _E_W // 16
_AR = 80
_NPAD = _AR * 128


def _sc_partials(u, src, dst, ra0, ra1, rows):
    mesh = plsc.VectorSubcoreMesh(
        core_axis_name="c", subcore_axis_name="s",
        num_cores=_NC, num_subcores=_NS)

    @functools.partial(
        pl.kernel,
        mesh=mesh,
        out_type=jax.ShapeDtypeStruct((_NC, 3, _AR, 128), jnp.float32),
        compiler_params=pltpu.CompilerParams(needs_layout_passes=False),
        scratch_types=[
            pltpu.VMEM((_N_NODES,), jnp.float32),
            pltpu.VMEM((_E_W,), jnp.int32),
            pltpu.VMEM((_E_W,), jnp.int32),
            pltpu.VMEM((_E_W,), jnp.float32),
            pltpu.VMEM((_E_W,), jnp.float32),
            pltpu.VMEM((_AR, 128), jnp.float32),
            pltpu.VMEM((_AR, 128), jnp.float32),
            pltpu.VMEM((_AR, 128), jnp.float32),
            pltpu.VMEM((1, _AR), jnp.int32),
            pltpu.VMEM_SHARED((_AR, 128), jnp.float32),
            pltpu.VMEM_SHARED((_AR, 128), jnp.float32),
            pltpu.VMEM_SHARED((_AR, 128), jnp.float32),
            pltpu.SemaphoreType.DMA,
        ],
    )
    def k(u_hbm, src_hbm, dst_hbm, ra0_hbm, ra1_hbm, rows_hbm, out_hbm,
          u_v, src_v, dst_v, ra0_v, ra1_v, accx, accy, accc, rows_v,
          shx, shy, shc, sem):
        cid = lax.axis_index("c")
        sid = lax.axis_index("s")
        base = (cid * _NS + sid) * _E_W

        d0 = pltpu.async_copy(u_hbm, u_v, sem)
        d1 = pltpu.async_copy(src_hbm.at[pl.ds(base, _E_W)], src_v, sem)
        d2 = pltpu.async_copy(dst_hbm.at[pl.ds(base, _E_W)], dst_v, sem)
        d3 = pltpu.async_copy(ra0_hbm.at[pl.ds(base, _E_W)], ra0_v, sem)
        d4 = pltpu.async_copy(ra1_hbm.at[pl.ds(base, _E_W)], ra1_v, sem)
        d5 = pltpu.async_copy(rows_hbm, rows_v, sem)

        z16 = jnp.zeros((16,), jnp.float32)

        @plsc.parallel_loop(0, _AR * 8, unroll=8)
        def _(i):
            r = lax.shift_right_logical(i, 3)
            o = lax.bitwise_and(i, 7) * 16
            accx[r, pl.ds(o, 16)] = z16
            accy[r, pl.ds(o, 16)] = z16
            accc[r, pl.ds(o, 16)] = z16

        @pl.when(sid == 0)
        def _():
            pltpu.sync_copy(accx, shx)
            pltpu.sync_copy(accy, shy)
            pltpu.sync_copy(accc, shc)

        plsc.subcore_barrier()
        d0.wait(); d1.wait(); d2.wait(); d3.wait(); d4.wait(); d5.wait()

        ones_f = jnp.ones((16,), jnp.float32)

        @plsc.parallel_loop(0, _VECS, unroll=8)
        def _(i):
            off = pl.multiple_of(i * 16, 16)
            ids = src_v[pl.ds(off, 16)]
            idd = dst_v[pl.ds(off, 16)]
            us = plsc.load_gather(u_v, [ids])
            ud = plsc.load_gather(u_v, [idd])
            du = ud - us
            a0 = ra0_v[pl.ds(off, 16)]
            a1 = ra1_v[pl.ds(off, 16)]
            r = lax.shift_right_logical(idd, 7)
            c = lax.bitwise_and(idd, 127)
            plsc.addupdate_scatter(accx, [r, c], du * a0)
            plsc.addupdate_scatter(accy, [r, c], du * a1)
            plsc.addupdate_scatter(accc, [r, c], ones_f)

        pltpu.sync_copy(accx, shx.at[rows_v.at[0]], add=True)
        pltpu.sync_copy(accy, shy.at[rows_v.at[0]], add=True)
        pltpu.sync_copy(accc, shc.at[rows_v.at[0]], add=True)

        plsc.subcore_barrier()

        @pl.when(sid == 0)
        def _():
            pltpu.sync_copy(shx, out_hbm.at[cid, 0])
            pltpu.sync_copy(shy, out_hbm.at[cid, 1])
            pltpu.sync_copy(shc, out_hbm.at[cid, 2])

    return k(u, src, dst, ra0, ra1, rows)


def _combine(parts):
    def ck(p_ref, o_ref):
        p = p_ref[...]
        s = p[0] + p[1]
        num = s[0:2]
        cnt = jnp.maximum(s[2:3], 1.0)
        o_ref[...] = num / cnt

    return pl.pallas_call(
        ck,
        out_shape=jax.ShapeDtypeStruct((2, _NPAD), jnp.float32),
    )(parts)


def kernel(x, edge_index, edge_attr):
    u = x[:, 0]
    rows = jnp.arange(_AR, dtype=jnp.int32).reshape(1, _AR)
    parts = _sc_partials(u, edge_index[0], edge_index[1],
                         1.0 / edge_attr[:, 0], 1.0 / edge_attr[:, 1], rows)
    parts = parts.reshape(_NC, 3, _NPAD)
    o = _combine(parts)
    return o[:, :_N_NODES].T


# speedup vs baseline: 117.9777x; 1.6869x over previous
import functools

import jax
import jax.numpy as jnp
from jax import lax
from jax.experimental import pallas as pl
from jax.experimental.pallas import tpu as pltpu
from jax.experimental.pallas import tpu_sc as plsc

_N_NODES = 10000
_N_EDGES = 320000
_NC = 2
_NS = 16
_NW = _NC * _NS
_E_W = 9984
_TAIL = _N_EDGES - _NW * _E_W
_CHUNK = 1664
_NCHUNKS = _E_W // _CHUNK
_CVECS = _CHUNK // 16
_AR = 80
_NPAD = _AR * 128


def _sc_partials(u, ei, attr_t, rows):
    mesh = plsc.VectorSubcoreMesh(
        core_axis_name="c", subcore_axis_name="s",
        num_cores=_NC, num_subcores=_NS)

    @functools.partial(
        pl.kernel,
        mesh=mesh,
        out_type=jax.ShapeDtypeStruct((_NC, 3, _AR, 128), jnp.float32),
        compiler_params=pltpu.CompilerParams(needs_layout_passes=False),
        scratch_types=[
            pltpu.VMEM((_N_NODES,), jnp.float32),
            pltpu.VMEM((2, _CHUNK), jnp.int32),
            pltpu.VMEM((2, _CHUNK), jnp.int32),
            pltpu.VMEM((4, _CHUNK), jnp.float32),
            pltpu.VMEM((4, _CHUNK), jnp.float32),
            pltpu.VMEM((_AR, 128), jnp.float32),
            pltpu.VMEM((_AR, 128), jnp.float32),
            pltpu.VMEM((_AR, 128), jnp.float32),
            pltpu.VMEM((1, _AR), jnp.int32),
            pltpu.VMEM_SHARED((_AR, 128), jnp.float32),
            pltpu.VMEM_SHARED((_AR, 128), jnp.float32),
            pltpu.VMEM_SHARED((_AR, 128), jnp.float32),
            pltpu.SemaphoreType.DMA,
            pltpu.SemaphoreType.DMA,
            pltpu.SemaphoreType.DMA,
        ],
    )
    def k(u_hbm, ei_hbm, at_hbm, rows_hbm, out_hbm,
          u_v, ei0, ei1, at0, at1, accx, accy, accc, rows_v,
          shx, shy, shc, sem_s, sem_a, sem_b):
        cid = lax.axis_index("c")
        sid = lax.axis_index("s")
        w = cid * _NS + sid
        base = w * _E_W

        ei_bufs = (ei0, ei1)
        at_bufs = (at0, at1)
        sems = (sem_a, sem_b)

        def start_chunk(j, slot):
            st = base + j * _CHUNK
            de = pltpu.async_copy(
                ei_hbm.at[:, pl.ds(st, _CHUNK)], ei_bufs[slot], sems[slot])
            da = pltpu.async_copy(
                at_hbm.at[:, pl.ds(st, _CHUNK)], at_bufs[slot], sems[slot])
            return de, da

        du_ = pltpu.async_copy(u_hbm, u_v, sem_s)
        dr_ = pltpu.async_copy(rows_hbm, rows_v, sem_s)
        pend = start_chunk(0, 0)

        z16 = jnp.zeros((16,), jnp.float32)

        @plsc.parallel_loop(0, _AR * 8, unroll=8)
        def _(i):
            r = lax.shift_right_logical(i, 3)
            o = lax.bitwise_and(i, 7) * 16
            accx[r, pl.ds(o, 16)] = z16
            accy[r, pl.ds(o, 16)] = z16
            accc[r, pl.ds(o, 16)] = z16

        @pl.when(sid == 0)
        def _():
            pltpu.sync_copy(accx, shx)
            pltpu.sync_copy(accy, shy)
            pltpu.sync_copy(accc, shc)

        plsc.subcore_barrier()
        du_.wait()
        dr_.wait()

        ones_f = jnp.ones((16,), jnp.float32)

        def process(eib, atb, nvecs):
            @plsc.parallel_loop(0, nvecs, unroll=8)
            def _(i):
                off = pl.multiple_of(i * 16, 16)
                ids = eib[0, pl.ds(off, 16)]
                idd = eib[1, pl.ds(off, 16)]
                us = plsc.load_gather(u_v, [ids])
                ud = plsc.load_gather(u_v, [idd])
                du = ud - us
                a0 = atb[0, pl.ds(off, 16)]
                a1 = atb[1, pl.ds(off, 16)]
                r = lax.shift_right_logical(idd, 7)
                c = lax.bitwise_and(idd, 127)
                plsc.addupdate_scatter(accx, [r, c], du / a0)
                plsc.addupdate_scatter(accy, [r, c], du / a1)
                plsc.addupdate_scatter(accc, [r, c], ones_f)

        for j in range(_NCHUNKS):
            slot = j % 2
            if j + 1 < _NCHUNKS:
                nxt = start_chunk(j + 1, (j + 1) % 2)
            pend[0].wait()
            pend[1].wait()
            process(ei_bufs[slot], at_bufs[slot], _CVECS)
            if j + 1 < _NCHUNKS:
                pend = nxt

        @pl.when(w == _NW - 1)
        def _():
            st = _NW * _E_W
            pltpu.sync_copy(ei_hbm.at[:, pl.ds(st, _TAIL)],
                            ei0.at[:, pl.ds(0, _TAIL)])
            pltpu.sync_copy(at_hbm.at[:, pl.ds(st, _TAIL)],
                            at0.at[:, pl.ds(0, _TAIL)])
            process(ei0, at0, _TAIL // 16)

        pltpu.sync_copy(accx, shx.at[rows_v.at[0]], add=True)
        pltpu.sync_copy(accy, shy.at[rows_v.at[0]], add=True)
        pltpu.sync_copy(accc, shc.at[rows_v.at[0]], add=True)

        plsc.subcore_barrier()

        @pl.when(sid == 0)
        def _():
            pltpu.sync_copy(shx, out_hbm.at[cid, 0])
            pltpu.sync_copy(shy, out_hbm.at[cid, 1])
            pltpu.sync_copy(shc, out_hbm.at[cid, 2])

    return k(u, ei, attr_t, rows)


def _combine(parts):
    def ck(p_ref, o_ref):
        p = p_ref[...]
        s = p[0] + p[1]
        num = s[0:2]
        cnt = jnp.maximum(s[2:3], 1.0)
        o_ref[...] = num / cnt

    return pl.pallas_call(
        ck,
        out_shape=jax.ShapeDtypeStruct((2, _NPAD), jnp.float32),
    )(parts)


def kernel(x, edge_index, edge_attr):
    u = x[:, 0]
    rows = jnp.arange(_AR, dtype=jnp.int32).reshape(1, _AR)
    parts = _sc_partials(u, edge_index, edge_attr.T, rows)
    parts = parts.reshape(_NC, 3, _NPAD)
    o = _combine(parts)
    return o[:, :_N_NODES].T


# speedup vs baseline: 127.8941x; 1.0841x over previous
import functools

import jax
import jax.numpy as jnp
from jax import lax
from jax.experimental import pallas as pl
from jax.experimental.pallas import tpu as pltpu
from jax.experimental.pallas import tpu_sc as plsc

_N_NODES = 10000
_N_EDGES = 320000
_NC = 2
_NS = 16
_NW = _NC * _NS
_E_W = 9984
_TAIL = _N_EDGES - _NW * _E_W
_CHUNK = 1664
_NCHUNKS = _E_W // _CHUNK
_CVECS = _CHUNK // 16
_AR = 80
_NPAD = _AR * 128


def _sc_partials(u, ei, attr_t, rows):
    mesh = plsc.VectorSubcoreMesh(
        core_axis_name="c", subcore_axis_name="s",
        num_cores=_NC, num_subcores=_NS)

    @functools.partial(
        pl.kernel,
        mesh=mesh,
        out_type=jax.ShapeDtypeStruct((_NC, 3, _AR, 128), jnp.float32),
        compiler_params=pltpu.CompilerParams(needs_layout_passes=False),
        scratch_types=[
            pltpu.VMEM((_N_NODES,), jnp.float32),
            pltpu.VMEM((2, _CHUNK), jnp.int32),
            pltpu.VMEM((2, _CHUNK), jnp.int32),
            pltpu.VMEM((4, _CHUNK), jnp.float32),
            pltpu.VMEM((4, _CHUNK), jnp.float32),
            pltpu.VMEM((_AR, 128), jnp.float32),
            pltpu.VMEM((_AR, 128), jnp.float32),
            pltpu.VMEM((_AR, 128), jnp.float32),
            pltpu.VMEM((1, _AR), jnp.int32),
            pltpu.VMEM_SHARED((_AR, 128), jnp.float32),
            pltpu.VMEM_SHARED((_AR, 128), jnp.float32),
            pltpu.VMEM_SHARED((_AR, 128), jnp.float32),
            pltpu.SemaphoreType.DMA,
            pltpu.SemaphoreType.DMA,
            pltpu.SemaphoreType.DMA,
        ],
    )
    def k(u_hbm, ei_hbm, at_hbm, rows_hbm, out_hbm,
          u_v, ei0, ei1, at0, at1, accx, accy, accc, rows_v,
          shx, shy, shc, sem_s, sem_a, sem_b):
        cid = lax.axis_index("c")
        sid = lax.axis_index("s")
        w = cid * _NS + sid
        base = w * _E_W

        ei_bufs = (ei0, ei1)
        at_bufs = (at0, at1)
        sems = (sem_a, sem_b)

        def start_chunk(j, slot):
            st = base + j * _CHUNK
            de = pltpu.async_copy(
                ei_hbm.at[:, pl.ds(st, _CHUNK)], ei_bufs[slot], sems[slot])
            da = pltpu.async_copy(
                at_hbm.at[:, pl.ds(st, _CHUNK)], at_bufs[slot], sems[slot])
            return de, da

        du_ = pltpu.async_copy(u_hbm, u_v, sem_s)
        dr_ = pltpu.async_copy(rows_hbm, rows_v, sem_s)
        pend = start_chunk(0, 0)

        z16 = jnp.zeros((16,), jnp.float32)

        @plsc.parallel_loop(0, _AR * 8, unroll=8)
        def _(i):
            r = lax.shift_right_logical(i, 3)
            o = lax.bitwise_and(i, 7) * 16
            accx[r, pl.ds(o, 16)] = z16
            accy[r, pl.ds(o, 16)] = z16
            accc[r, pl.ds(o, 16)] = z16

        @pl.when(sid == 0)
        def _():
            pltpu.sync_copy(accx, shx)
            pltpu.sync_copy(accy, shy)
            pltpu.sync_copy(accc, shc)

        plsc.subcore_barrier()
        du_.wait()
        dr_.wait()

        ones_f = jnp.ones((16,), jnp.float32)

        def process(eib, atb, nvecs):
            @plsc.parallel_loop(0, nvecs, unroll=8)
            def _(i):
                off = pl.multiple_of(i * 16, 16)
                ids = eib[0, pl.ds(off, 16)]
                idd = eib[1, pl.ds(off, 16)]
                us = plsc.load_gather(u_v, [ids])
                ud = plsc.load_gather(u_v, [idd])
                du = ud - us
                a0 = atb[0, pl.ds(off, 16)]
                a1 = atb[1, pl.ds(off, 16)]
                r = lax.shift_right_logical(idd, 7)
                c = lax.bitwise_and(idd, 127)
                plsc.addupdate_scatter(accx, [r, c], du / a0)
                plsc.addupdate_scatter(accy, [r, c], du / a1)
                plsc.addupdate_scatter(accc, [r, c], ones_f)

        for j in range(_NCHUNKS):
            slot = j % 2
            if j + 1 < _NCHUNKS:
                nxt = start_chunk(j + 1, (j + 1) % 2)
            pend[0].wait()
            pend[1].wait()
            process(ei_bufs[slot], at_bufs[slot], _CVECS)
            if j + 1 < _NCHUNKS:
                pend = nxt

        @pl.when(w == _NW - 1)
        def _():
            st = _NW * _E_W
            pltpu.sync_copy(ei_hbm.at[:, pl.ds(st, _TAIL)],
                            ei0.at[:, pl.ds(0, _TAIL)])
            pltpu.sync_copy(at_hbm.at[:, pl.ds(st, _TAIL)],
                            at0.at[:, pl.ds(0, _TAIL)])
            process(ei0, at0, _TAIL // 16)

        pltpu.sync_copy(accx, shx.at[rows_v.at[0]], add=True)
        pltpu.sync_copy(accy, shy.at[rows_v.at[0]], add=True)
        pltpu.sync_copy(accc, shc.at[rows_v.at[0]], add=True)

        plsc.subcore_barrier()

        @pl.when(sid == 0)
        def _():
            pltpu.sync_copy(shx, out_hbm.at[cid, 0])
            pltpu.sync_copy(shy, out_hbm.at[cid, 1])
            pltpu.sync_copy(shc, out_hbm.at[cid, 2])

    return k(u, ei, attr_t, rows)


def _combine(parts):
    def ck(p_ref, o_ref):
        p = p_ref[...]
        s = p[0] + p[1]
        num = s[0:2].reshape(2, _NPAD)
        cnt = jnp.maximum(s[2].reshape(1, _NPAD), 1.0)
        o_ref[...] = (num / cnt)[:, :_N_NODES]

    return pl.pallas_call(
        ck,
        out_shape=jax.ShapeDtypeStruct((2, _N_NODES), jnp.float32),
    )(parts)


def kernel(x, edge_index, edge_attr):
    u = x[:, 0]
    rows = jnp.arange(_AR, dtype=jnp.int32).reshape(1, _AR)
    parts = _sc_partials(u, edge_index, edge_attr.T, rows)
    o = _combine(parts)
    return o.T


# speedup vs baseline: 130.9420x; 1.0238x over previous
import functools

import jax
import jax.numpy as jnp
from jax import lax
from jax.experimental import pallas as pl
from jax.experimental.pallas import tpu as pltpu
from jax.experimental.pallas import tpu_sc as plsc

_N_NODES = 10000
_N_EDGES = 320000
_NC = 2
_NS = 16
_NW = _NC * _NS
_E_W = 9984
_TAIL = _N_EDGES - _NW * _E_W
_CHUNK = 1664
_NCHUNKS = _E_W // _CHUNK
_CVECS = _CHUNK // 16
_AR = 80
_NPAD = _AR * 128


def _sc_partials(u, ei, attr_t, rows):
    mesh = plsc.VectorSubcoreMesh(
        core_axis_name="c", subcore_axis_name="s",
        num_cores=_NC, num_subcores=_NS)

    @functools.partial(
        pl.kernel,
        mesh=mesh,
        out_type=jax.ShapeDtypeStruct((_NC, 3, _AR, 128), jnp.float32),
        compiler_params=pltpu.CompilerParams(needs_layout_passes=False),
        scratch_types=[
            pltpu.VMEM((_N_NODES,), jnp.float32),
            pltpu.VMEM((2, _CHUNK), jnp.int32),
            pltpu.VMEM((2, _CHUNK), jnp.int32),
            pltpu.VMEM((4, _CHUNK), jnp.float32),
            pltpu.VMEM((4, _CHUNK), jnp.float32),
            pltpu.VMEM((_AR, 128), jnp.float32),
            pltpu.VMEM((_AR, 128), jnp.float32),
            pltpu.VMEM((_AR, 128), jnp.float32),
            pltpu.VMEM((1, _AR), jnp.int32),
            pltpu.VMEM_SHARED((_AR, 128), jnp.float32),
            pltpu.VMEM_SHARED((_AR, 128), jnp.float32),
            pltpu.VMEM_SHARED((_AR, 128), jnp.float32),
            pltpu.SemaphoreType.DMA,
            pltpu.SemaphoreType.DMA,
            pltpu.SemaphoreType.DMA,
        ],
    )
    def k(u_hbm, ei_hbm, at_hbm, rows_hbm, out_hbm,
          u_v, ei0, ei1, at0, at1, accx, accy, accc, rows_v,
          shx, shy, shc, sem_s, sem_a, sem_b):
        cid = lax.axis_index("c")
        sid = lax.axis_index("s")
        w = cid * _NS + sid
        base = w * _E_W

        ei_bufs = (ei0, ei1)
        at_bufs = (at0, at1)
        sems = (sem_a, sem_b)

        def start_chunk(j, slot):
            st = pl.multiple_of(base + j * _CHUNK, 128)
            de = pltpu.async_copy(
                ei_hbm.at[:, pl.ds(st, _CHUNK)], ei_bufs[slot], sems[slot])
            da = pltpu.async_copy(
                at_hbm.at[:, pl.ds(st, _CHUNK)], at_bufs[slot], sems[slot])
            return de, da

        def wait_chunk(slot):
            pltpu.make_async_copy(
                ei_hbm.at[:, pl.ds(0, _CHUNK)], ei_bufs[slot],
                sems[slot]).wait()
            pltpu.make_async_copy(
                at_hbm.at[:, pl.ds(0, _CHUNK)], at_bufs[slot],
                sems[slot]).wait()

        du_ = pltpu.async_copy(u_hbm, u_v, sem_s)
        dr_ = pltpu.async_copy(rows_hbm, rows_v, sem_s)
        start_chunk(0, 0)

        z16 = jnp.zeros((16,), jnp.float32)

        @plsc.parallel_loop(0, _AR * 8, unroll=8)
        def _(i):
            r = lax.shift_right_logical(i, 3)
            o = lax.bitwise_and(i, 7) * 16
            accx[r, pl.ds(o, 16)] = z16
            accy[r, pl.ds(o, 16)] = z16
            accc[r, pl.ds(o, 16)] = z16

        @pl.when(sid == 0)
        def _():
            pltpu.sync_copy(accx, shx)
            pltpu.sync_copy(accy, shy)
            pltpu.sync_copy(accc, shc)

        plsc.subcore_barrier()
        du_.wait()
        dr_.wait()

        ones_f = jnp.ones((16,), jnp.float32)

        def process(eib, atb, nvecs):
            @plsc.parallel_loop(0, nvecs, unroll=4)
            def _(i):
                off = pl.multiple_of(i * 16, 16)
                ids = eib[0, pl.ds(off, 16)]
                idd = eib[1, pl.ds(off, 16)]
                us = plsc.load_gather(u_v, [ids])
                ud = plsc.load_gather(u_v, [idd])
                du = ud - us
                a0 = atb[0, pl.ds(off, 16)]
                a1 = atb[1, pl.ds(off, 16)]
                r = lax.shift_right_logical(idd, 7)
                c = lax.bitwise_and(idd, 127)
                plsc.addupdate_scatter(accx, [r, c], du / a0)
                plsc.addupdate_scatter(accy, [r, c], du / a1)
                plsc.addupdate_scatter(accc, [r, c], ones_f)

        start_chunk(1, 1)

        @pl.loop(0, _NCHUNKS // 2)
        def _(j):
            for b in range(2):
                cidx = j * 2 + b
                wait_chunk(b)
                process(ei_bufs[b], at_bufs[b], _CVECS)

                @pl.when(cidx + 2 < _NCHUNKS)
                def _():
                    start_chunk(cidx + 2, b)

        @pl.when(w == _NW - 1)
        def _():
            st = _NW * _E_W
            pltpu.sync_copy(ei_hbm.at[:, pl.ds(st, _TAIL)],
                            ei0.at[:, pl.ds(0, _TAIL)])
            pltpu.sync_copy(at_hbm.at[:, pl.ds(st, _TAIL)],
                            at0.at[:, pl.ds(0, _TAIL)])
            process(ei0, at0, _TAIL // 16)

        pltpu.sync_copy(accx, shx.at[rows_v.at[0]], add=True)
        pltpu.sync_copy(accy, shy.at[rows_v.at[0]], add=True)
        pltpu.sync_copy(accc, shc.at[rows_v.at[0]], add=True)

        plsc.subcore_barrier()

        @pl.when(sid == 0)
        def _():
            pltpu.sync_copy(shx, out_hbm.at[cid, 0])
            pltpu.sync_copy(shy, out_hbm.at[cid, 1])
            pltpu.sync_copy(shc, out_hbm.at[cid, 2])

    return k(u, ei, attr_t, rows)


def _combine(parts):
    def ck(p_ref, o_ref):
        p = p_ref[...]
        s = p[0] + p[1]
        num = s[0:2].reshape(2, _NPAD)
        cnt = jnp.maximum(s[2].reshape(1, _NPAD), 1.0)
        o_ref[...] = (num / cnt)[:, :_N_NODES]

    return pl.pallas_call(
        ck,
        out_shape=jax.ShapeDtypeStruct((2, _N_NODES), jnp.float32),
    )(parts)


def kernel(x, edge_index, edge_attr):
    u = x[:, 0]
    rows = jnp.arange(_AR, dtype=jnp.int32).reshape(1, _AR)
    parts = _sc_partials(u, edge_index, edge_attr.T, rows)
    o = _combine(parts)
    return o.T


# speedup vs baseline: 133.4167x; 1.0189x over previous
import functools

import jax
import jax.numpy as jnp
from jax import lax
from jax.experimental import pallas as pl
from jax.experimental.pallas import tpu as pltpu
from jax.experimental.pallas import tpu_sc as plsc

_N_NODES = 10000
_N_EDGES = 320000
_NC = 2
_NS = 16
_NW = _NC * _NS
_E_W = 9984
_TAIL = _N_EDGES - _NW * _E_W
_CHUNK = 1664
_NCHUNKS = _E_W // _CHUNK
_CVECS = _CHUNK // 16
_AR = 80
_NPAD = _AR * 128


def _sc_partials(u, ei, attr_t):
    mesh = plsc.VectorSubcoreMesh(
        core_axis_name="c", subcore_axis_name="s",
        num_cores=_NC, num_subcores=_NS)

    @functools.partial(
        pl.kernel,
        mesh=mesh,
        out_type=jax.ShapeDtypeStruct((_NC, 3, _AR, 128), jnp.float32),
        compiler_params=pltpu.CompilerParams(needs_layout_passes=False),
        scratch_types=[
            pltpu.VMEM((_N_NODES,), jnp.float32),
            pltpu.VMEM((2, _CHUNK), jnp.int32),
            pltpu.VMEM((2, _CHUNK), jnp.int32),
            pltpu.VMEM((4, _CHUNK), jnp.float32),
            pltpu.VMEM((4, _CHUNK), jnp.float32),
            pltpu.VMEM((_AR, 128), jnp.float32),
            pltpu.VMEM((_AR, 128), jnp.float32),
            pltpu.VMEM((_AR, 128), jnp.float32),
            pltpu.VMEM((1, _AR), jnp.int32),
            pltpu.VMEM_SHARED((_AR, 128), jnp.float32),
            pltpu.VMEM_SHARED((_AR, 128), jnp.float32),
            pltpu.VMEM_SHARED((_AR, 128), jnp.float32),
            pltpu.SemaphoreType.DMA,
            pltpu.SemaphoreType.DMA,
            pltpu.SemaphoreType.DMA,
        ],
    )
    def k(u_hbm, ei_hbm, at_hbm, out_hbm,
          u_v, ei0, ei1, at0, at1, accx, accy, accc, rows_v,
          shx, shy, shc, sem_s, sem_a, sem_b):
        cid = lax.axis_index("c")
        sid = lax.axis_index("s")
        w = cid * _NS + sid
        base = w * _E_W

        ei_bufs = (ei0, ei1)
        at_bufs = (at0, at1)
        sems = (sem_a, sem_b)

        def start_chunk(j, slot):
            st = pl.multiple_of(base + j * _CHUNK, 128)
            de = pltpu.async_copy(
                ei_hbm.at[:, pl.ds(st, _CHUNK)], ei_bufs[slot], sems[slot])
            da = pltpu.async_copy(
                at_hbm.at[:, pl.ds(st, _CHUNK)], at_bufs[slot], sems[slot])
            return de, da

        def wait_chunk(slot):
            pltpu.make_async_copy(
                ei_hbm.at[:, pl.ds(0, _CHUNK)], ei_bufs[slot],
                sems[slot]).wait()
            pltpu.make_async_copy(
                at_hbm.at[:, pl.ds(0, _CHUNK)], at_bufs[slot],
                sems[slot]).wait()

        du_ = pltpu.async_copy(u_hbm, u_v, sem_s)
        start_chunk(0, 0)

        iota16 = lax.iota(jnp.int32, 16)
        for kk in range(_AR // 16):
            rows_v[0, pl.ds(kk * 16, 16)] = iota16 + (kk * 16)

        z16 = jnp.zeros((16,), jnp.float32)

        @plsc.parallel_loop(0, _AR * 8, unroll=8)
        def _(i):
            r = lax.shift_right_logical(i, 3)
            o = lax.bitwise_and(i, 7) * 16
            accx[r, pl.ds(o, 16)] = z16
            accy[r, pl.ds(o, 16)] = z16
            accc[r, pl.ds(o, 16)] = z16

        @pl.when(sid == 0)
        def _():
            pltpu.sync_copy(accx, shx)
            pltpu.sync_copy(accy, shy)
            pltpu.sync_copy(accc, shc)

        plsc.subcore_barrier()
        du_.wait()

        ones_f = jnp.ones((16,), jnp.float32)

        def process(eib, atb, nvecs):
            @plsc.parallel_loop(0, nvecs, unroll=8)
            def _(i):
                off = pl.multiple_of(i * 16, 16)
                ids = eib[0, pl.ds(off, 16)]
                idd = eib[1, pl.ds(off, 16)]
                us = plsc.load_gather(u_v, [ids])
                ud = plsc.load_gather(u_v, [idd])
                du = ud - us
                a0 = atb[0, pl.ds(off, 16)]
                a1 = atb[1, pl.ds(off, 16)]
                r = lax.shift_right_logical(idd, 7)
                c = lax.bitwise_and(idd, 127)
                plsc.addupdate_scatter(accx, [r, c], du / a0)
                plsc.addupdate_scatter(accy, [r, c], du / a1)
                plsc.addupdate_scatter(accc, [r, c], ones_f)

        start_chunk(1, 1)

        @pl.loop(0, _NCHUNKS // 2)
        def _(j):
            for b in range(2):
                cidx = j * 2 + b
                wait_chunk(b)
                process(ei_bufs[b], at_bufs[b], _CVECS)

                @pl.when(cidx + 2 < _NCHUNKS)
                def _():
                    start_chunk(cidx + 2, b)

        @pl.when(w == _NW - 1)
        def _():
            st = _NW * _E_W
            pltpu.sync_copy(ei_hbm.at[:, pl.ds(st, _TAIL)],
                            ei0.at[:, pl.ds(0, _TAIL)])
            pltpu.sync_copy(at_hbm.at[:, pl.ds(st, _TAIL)],
                            at0.at[:, pl.ds(0, _TAIL)])
            process(ei0, at0, _TAIL // 16)

        pltpu.sync_copy(accx, shx.at[rows_v.at[0]], add=True)
        pltpu.sync_copy(accy, shy.at[rows_v.at[0]], add=True)
        pltpu.sync_copy(accc, shc.at[rows_v.at[0]], add=True)

        plsc.subcore_barrier()

        @pl.when(sid == 0)
        def _():
            pltpu.sync_copy(shx, out_hbm.at[cid, 0])
            pltpu.sync_copy(shy, out_hbm.at[cid, 1])
            pltpu.sync_copy(shc, out_hbm.at[cid, 2])

    return k(u, ei, attr_t)


def _combine(parts):
    def ck(p_ref, o_ref):
        p = p_ref[...]
        s = p[0] + p[1]
        num = s[0:2].reshape(2, _NPAD)
        cnt = jnp.maximum(s[2].reshape(1, _NPAD), 1.0)
        o_ref[...] = (num / cnt)[:, :_N_NODES]

    return pl.pallas_call(
        ck,
        out_shape=jax.ShapeDtypeStruct((2, _N_NODES), jnp.float32),
    )(parts)


def kernel(x, edge_index, edge_attr):
    u = x[:, 0]
    parts = _sc_partials(u, edge_index, edge_attr.T)
    o = _combine(parts)
    return o.T
